# baseline (speedup 1.0000x reference)
import functools

import jax
import jax.numpy as jnp
from jax import lax
from jax.experimental import pallas as pl
from jax.experimental.pallas import tpu as pltpu
from jax.experimental.pallas import tpu_sc as plsc

NUM_CORES = 2
NUM_SUBCORES = 16
NUM_WORKERS = NUM_CORES * NUM_SUBCORES
LANES = 16
CHUNK = 80

_GATHER_DN = lax.GatherDimensionNumbers(
    offset_dims=(), collapsed_slice_dims=(0,), start_index_map=(0,))


def _vgather(v, idx):
    return lax.gather(v, idx[:, None], _GATHER_DN, slice_sizes=(1,),
                      mode=lax.GatherScatterMode.PROMISE_IN_BOUNDS)


def _scale_body(x_ref, w_ref, xb_ref, y_ref):
    xv = x_ref[...]
    z = jnp.zeros(xv.shape, jnp.bfloat16)
    xb_ref[...] = jnp.concatenate([xv.astype(jnp.bfloat16), z], axis=1)
    y_ref[...] = jnp.concatenate(
        [(xv * w_ref[...]).astype(jnp.bfloat16), z], axis=1)


def _scale_rows(x, w_row):
    n, d = x.shape
    blk = 1000
    return pl.pallas_call(
        _scale_body,
        out_shape=[jax.ShapeDtypeStruct((n, 2 * d), jnp.bfloat16),
                   jax.ShapeDtypeStruct((n, 2 * d), jnp.bfloat16)],
        grid=(n // blk,),
        in_specs=[pl.BlockSpec((blk, d), lambda i: (i, 0)),
                  pl.BlockSpec((1, d), lambda i: (0, 0))],
        out_specs=[pl.BlockSpec((blk, 2 * d), lambda i: (i, 0)),
                   pl.BlockSpec((blk, 2 * d), lambda i: (i, 0))],
    )(x, w_row)


def _sc_body(x_hbm, y_hbm, src_hbm, dst_hbm, b_hbm, out_hbm,
             b_v, src_ix, dst_ix, out_all,
             srows0, drows0, srows1, drows1, sem0, sem1):
    n_blk = x_hbm.shape[1] // (2 * LANES)
    wid = lax.axis_index("s") * NUM_CORES + lax.axis_index("c")
    e_per_w = src_hbm.shape[0] // NUM_WORKERS
    base = wid * e_per_w
    n_chunks = e_per_w // CHUNK

    pltpu.sync_copy(b_hbm, b_v)
    pltpu.sync_copy(src_hbm.at[pl.ds(base, e_per_w)], src_ix)
    pltpu.sync_copy(dst_hbm.at[pl.ds(base, e_per_w)], dst_ix)

    b_vec = b_v[...]
    lane_iota = lax.iota(jnp.int32, LANES)

    def issue(i, srows, drows, sem):
        sl = pl.ds(i * CHUNK, CHUNK)
        pltpu.async_copy(x_hbm.at[src_ix.at[sl]], srows, sem)
        pltpu.async_copy(y_hbm.at[dst_ix.at[sl]], drows, sem)

    def wait(i, srows, drows, sem):
        sl = pl.ds(i * CHUNK, CHUNK)
        pltpu.make_async_copy(x_hbm.at[src_ix.at[sl]], srows, sem).wait()
        pltpu.make_async_copy(y_hbm.at[dst_ix.at[sl]], drows, sem).wait()

    def compute(i, srows, drows):
        obase = i * CHUNK
        sbf = srows.bitcast(jnp.bfloat16)
        dbf = drows.bitcast(jnp.bfloat16)

        def group_body(g, c):
            ebase = g * LANES
            init = tuple(sbf[2 * (ebase + j), pl.ds(0, 2 * LANES)]
                         * dbf[2 * (ebase + j), pl.ds(0, 2 * LANES)]
                         for j in range(LANES))

            def kbody(k, accs):
                off = k * 2 * LANES
                return tuple(
                    accs[j] + (sbf[2 * (ebase + j), pl.ds(off, 2 * LANES)]
                               * dbf[2 * (ebase + j), pl.ds(off, 2 * LANES)])
                    for j in range(LANES))

            accs = lax.fori_loop(1, n_blk, kbody, init, unroll=False)
            accs = [lax.slice(a, (0,), (LANES,)).astype(jnp.float32)
                    + lax.slice(a, (LANES,), (2 * LANES,)).astype(jnp.float32)
                    for a in accs]
            for dd in (1, 2, 4, 8):
                m = (lane_iota & dd) != 0
                rot_idx = lane_iota ^ dd
                nxt = []
                for t in range(0, len(accs), 2):
                    a, bb = accs[t], accs[t + 1]
                    sel = jnp.where(m, bb, a)
                    rot = _vgather(jnp.where(m, a, bb), rot_idx)
                    nxt.append(sel + rot)
                accs = nxt
            out_all[pl.ds(obase + ebase, LANES)] = accs[0] + b_vec
            return c

        lax.fori_loop(0, CHUNK // LANES, group_body, 0, unroll=False)

    issue(0, srows0, drows0, sem0)

    def pair_body(p, carry):
        c0 = 2 * p
        c1 = c0 + 1
        issue(c1, srows1, drows1, sem1)
        wait(c0, srows0, drows0, sem0)
        compute(c0, srows0, drows0)

        @pl.when(c1 + 1 < n_chunks)
        def _():
            issue(c1 + 1, srows0, drows0, sem0)

        wait(c1, srows1, drows1, sem1)
        compute(c1, srows1, drows1)
        return carry

    lax.fori_loop(0, n_chunks // 2, pair_body, 0, unroll=False)

    if n_chunks % 2 == 1:
        wait(n_chunks - 1, srows0, drows0, sem0)
        compute(n_chunks - 1, srows0, drows0)

    pltpu.sync_copy(out_all, out_hbm.at[pl.ds(base, e_per_w)])


def _make_sc_call(n_edges, dp):
    mesh = plsc.VectorSubcoreMesh(core_axis_name="c", subcore_axis_name="s")
    e_per_w = n_edges // NUM_WORKERS
    return pl.kernel(
        _sc_body,
        out_type=jax.ShapeDtypeStruct((n_edges,), jnp.float32),
        mesh=mesh,
        scratch_types=[
            pltpu.VMEM((LANES,), jnp.float32),
            pltpu.VMEM((e_per_w,), jnp.int32),
            pltpu.VMEM((e_per_w,), jnp.int32),
            pltpu.VMEM((e_per_w,), jnp.float32),
            pltpu.VMEM((CHUNK, dp), jnp.int32),
            pltpu.VMEM((CHUNK, dp), jnp.int32),
            pltpu.VMEM((CHUNK, dp), jnp.int32),
            pltpu.VMEM((CHUNK, dp), jnp.int32),
            pltpu.SemaphoreType.DMA,
            pltpu.SemaphoreType.DMA,
        ],
    )


def _pack(t_bf16):
    n, d2 = t_bf16.shape
    return lax.bitcast_convert_type(
        t_bf16.reshape(n, d2 // 2, 2), jnp.int32)


def kernel(x, edge_index, W, b):
    n_edges = edge_index.shape[1]
    d = x.shape[1]
    src = edge_index[0].astype(jnp.int32)
    dst = edge_index[1].astype(jnp.int32)
    xb, y = _scale_rows(x, W.astype(jnp.float32))
    b16 = jnp.broadcast_to(b.astype(jnp.float32), (LANES,))
    out = _make_sc_call(n_edges, d)(_pack(xb), _pack(y), src, dst, b16)
    return out.reshape(n_edges, 1)


# speedup vs baseline: 2.0727x; 2.0727x over previous
import functools

import jax
import jax.numpy as jnp
from jax import lax
from jax.experimental import pallas as pl
from jax.experimental.pallas import tpu as pltpu
from jax.experimental.pallas import tpu_sc as plsc

NUM_CORES = 2
NUM_SUBCORES = 16
NUM_WORKERS = NUM_CORES * NUM_SUBCORES
LANES = 16
CHUNK = 80

_GATHER_DN = lax.GatherDimensionNumbers(
    offset_dims=(), collapsed_slice_dims=(0,), start_index_map=(0,))


def _vgather(v, idx):
    return lax.gather(v, idx[:, None], _GATHER_DN, slice_sizes=(1,),
                      mode=lax.GatherScatterMode.PROMISE_IN_BOUNDS)


def _sc_body(x_hbm, src_hbm, dst_hbm, wb_hbm, out_hbm,
             wb_v, src_ix, dst_ix, out_all,
             srows0, drows0, srows1, drows1, sem0, sem1):
    d = x_hbm.shape[1]
    n_sub = d // LANES
    wid = lax.axis_index("s") * NUM_CORES + lax.axis_index("c")
    e_per_w = src_hbm.shape[0] // NUM_WORKERS
    base = wid * e_per_w
    n_chunks = e_per_w // CHUNK

    pltpu.sync_copy(wb_hbm, wb_v)
    pltpu.sync_copy(src_hbm.at[pl.ds(base, e_per_w)], src_ix)
    pltpu.sync_copy(dst_hbm.at[pl.ds(base, e_per_w)], dst_ix)

    b_vec = wb_v[pl.ds(d, LANES)]
    lane_iota = lax.iota(jnp.int32, LANES)

    def issue(i, srows, drows, sem):
        sl = pl.ds(i * CHUNK, CHUNK)
        pltpu.async_copy(x_hbm.at[src_ix.at[sl]], srows, sem)
        pltpu.async_copy(x_hbm.at[dst_ix.at[sl]], drows, sem)

    def wait(i, srows, drows, sem):
        sl = pl.ds(i * CHUNK, CHUNK)
        pltpu.make_async_copy(x_hbm.at[src_ix.at[sl]], srows, sem).wait()
        pltpu.make_async_copy(x_hbm.at[dst_ix.at[sl]], drows, sem).wait()

    def compute(i, srows, drows):
        obase = i * CHUNK

        def group_body(g, c):
            ebase = g * LANES
            w0 = wb_v[pl.ds(0, LANES)]
            init = tuple(srows[ebase + j, pl.ds(0, LANES)]
                         * drows[ebase + j, pl.ds(0, LANES)] * w0
                         for j in range(LANES))

            def kbody(k, accs):
                off = k * LANES
                wk = wb_v[pl.ds(off, LANES)]
                return tuple(
                    accs[j] + (srows[ebase + j, pl.ds(off, LANES)]
                               * drows[ebase + j, pl.ds(off, LANES)]) * wk
                    for j in range(LANES))

            accs = list(lax.fori_loop(1, n_sub, kbody, init, unroll=False))
            for dd in (1, 2, 4, 8):
                m = (lane_iota & dd) != 0
                rot_idx = lane_iota ^ dd
                nxt = []
                for t in range(0, len(accs), 2):
                    a, bb = accs[t], accs[t + 1]
                    sel = jnp.where(m, bb, a)
                    rot = _vgather(jnp.where(m, a, bb), rot_idx)
                    nxt.append(sel + rot)
                accs = nxt
            out_all[pl.ds(obase + ebase, LANES)] = accs[0] + b_vec
            return c

        lax.fori_loop(0, CHUNK // LANES, group_body, 0, unroll=False)

    issue(0, srows0, drows0, sem0)

    def pair_body(p, carry):
        c0 = 2 * p
        c1 = c0 + 1
        issue(c1, srows1, drows1, sem1)
        wait(c0, srows0, drows0, sem0)
        compute(c0, srows0, drows0)

        @pl.when(c1 + 1 < n_chunks)
        def _():
            issue(c1 + 1, srows0, drows0, sem0)

        wait(c1, srows1, drows1, sem1)
        compute(c1, srows1, drows1)
        return carry

    lax.fori_loop(0, n_chunks // 2, pair_body, 0, unroll=False)

    if n_chunks % 2 == 1:
        wait(n_chunks - 1, srows0, drows0, sem0)
        compute(n_chunks - 1, srows0, drows0)

    pltpu.sync_copy(out_all, out_hbm.at[pl.ds(base, e_per_w)])


def _make_sc_call(n_edges, d):
    mesh = plsc.VectorSubcoreMesh(core_axis_name="c", subcore_axis_name="s")
    e_per_w = n_edges // NUM_WORKERS
    return pl.kernel(
        _sc_body,
        out_type=jax.ShapeDtypeStruct((n_edges,), jnp.float32),
        mesh=mesh,
        scratch_types=[
            pltpu.VMEM((d + LANES,), jnp.float32),
            pltpu.VMEM((e_per_w,), jnp.int32),
            pltpu.VMEM((e_per_w,), jnp.int32),
            pltpu.VMEM((e_per_w,), jnp.float32),
            pltpu.VMEM((CHUNK, d), jnp.float32),
            pltpu.VMEM((CHUNK, d), jnp.float32),
            pltpu.VMEM((CHUNK, d), jnp.float32),
            pltpu.VMEM((CHUNK, d), jnp.float32),
            pltpu.SemaphoreType.DMA,
            pltpu.SemaphoreType.DMA,
        ],
    )


def kernel(x, edge_index, W, b):
    n_edges = edge_index.shape[1]
    d = x.shape[1]
    src = edge_index[0].astype(jnp.int32)
    dst = edge_index[1].astype(jnp.int32)
    wb = jnp.concatenate(
        [W[0].astype(jnp.float32),
         jnp.broadcast_to(b.astype(jnp.float32), (LANES,))])
    out = _make_sc_call(n_edges, d)(x, src, dst, wb)
    return out.reshape(n_edges, 1)
